# Initial kernel scaffold; baseline (speedup 1.0000x reference)
import jax
import jax.numpy as jnp
from jax.experimental import pallas as pl


def kernel(customer_id, category_by_Gender, user_table, gender_table, W1, b1, W2, b2, W3, b3):
    raise NotImplementedError("write your pallas kernel here")


# speedup vs baseline: 1.6057x; 1.6057x over previous
import functools

import jax
import jax.numpy as jnp
from jax import lax
from jax.experimental import pallas as pl
from jax.experimental.pallas import tpu as pltpu
from jax.experimental.pallas import tpu_sc as plsc

B = 16384
EMB = 32
GPAD = 16
BLK = 2048
GRID = B // BLK


@functools.cache
def _build_gather():
    info = plsc.get_sparse_core_info()
    nc, ns = info.num_cores, info.num_subcores
    nw = nc * ns
    b_per_w = B // nw
    mesh = plsc.VectorSubcoreMesh(core_axis_name="c", subcore_axis_name="s")

    @functools.partial(
        pl.kernel,
        mesh=mesh,
        out_type=jax.ShapeDtypeStruct((B, EMB), jnp.float32),
        scratch_types=[
            pltpu.VMEM((b_per_w,), jnp.int32),
            pltpu.VMEM((b_per_w, EMB), jnp.float32),
            pltpu.SemaphoreType.DMA,
        ],
        compiler_params=pltpu.CompilerParams(use_tc_tiling_on_sc=False),
    )
    def gather(table_hbm, idx_hbm, out_hbm, idx_v, rows_v, sem):
        wid = lax.axis_index("s") * nc + lax.axis_index("c")
        base = wid * b_per_w
        pltpu.sync_copy(idx_hbm.at[pl.ds(base, b_per_w)], idx_v)
        pltpu.async_copy(table_hbm.at[idx_v], rows_v, sem).wait()
        pltpu.sync_copy(rows_v, out_hbm.at[pl.ds(base, b_per_w)])

    return gather


def _mlp_body(cat_ref, u_ref, gt_ref, w1_ref, b1_ref, w2_ref, b2_ref,
              w3_ref, b3_ref, out_ref):
    u = u_ref[...]
    cat = cat_ref[0, 0, :]
    col = lax.broadcasted_iota(jnp.int32, (BLK, GPAD), 1)
    onehot = (col == cat[:, None]).astype(jnp.float32)
    g = jnp.dot(onehot, gt_ref[...], preferred_element_type=jnp.float32)
    x = jnp.concatenate([u, g], axis=1)
    h = jnp.maximum(
        jnp.dot(x, w1_ref[...], preferred_element_type=jnp.float32)
        + b1_ref[...], 0.0)
    h = jnp.maximum(
        jnp.dot(h, w2_ref[...], preferred_element_type=jnp.float32)
        + b2_ref[...], 0.0)
    out_ref[...] = (
        jnp.dot(h, w3_ref[...], preferred_element_type=jnp.float32)
        + b3_ref[...])


@functools.cache
def _build_mlp(interpret=False):
    full = lambda *shape: pl.BlockSpec(shape, lambda i: (0,) * len(shape))
    return pl.pallas_call(
        _mlp_body,
        grid=(GRID,),
        in_specs=[
            pl.BlockSpec((1, 1, BLK), lambda i: (i, 0, 0)),
            pl.BlockSpec((BLK, EMB), lambda i: (i, 0)),
            full(GPAD, EMB),
            full(2 * EMB, 128), full(1, 128),
            full(128, 64), full(1, 64),
            full(64, EMB), full(1, EMB),
        ],
        out_specs=pl.BlockSpec((BLK, EMB), lambda i: (i, 0)),
        out_shape=jax.ShapeDtypeStruct((B, EMB), jnp.float32),
        interpret=interpret,
    )


def kernel(customer_id, category_by_Gender, user_table, gender_table,
           W1, b1, W2, b2, W3, b3):
    cid = customer_id.astype(jnp.int32)
    cat = category_by_Gender.astype(jnp.int32).reshape(GRID, 1, BLK)
    user_emb = _build_gather()(user_table, cid)
    gt_pad = jnp.pad(gender_table, ((0, GPAD - gender_table.shape[0]), (0, 0)))
    return _build_mlp()(
        cat, user_emb, gt_pad,
        W1, b1.reshape(1, -1), W2, b2.reshape(1, -1), W3, b3.reshape(1, -1))


# speedup vs baseline: 1.8394x; 1.1456x over previous
import functools

import jax
import jax.numpy as jnp
from jax import lax
from jax.experimental import pallas as pl
from jax.experimental.pallas import tpu as pltpu
from jax.experimental.pallas import tpu_sc as plsc

B = 16384
EMB = 32
ROWP = 128
GPAD = 16
BLK = 2048
GRID = B // BLK


@functools.cache
def _build_gather():
    info = plsc.get_sparse_core_info()
    nc, ns = info.num_cores, info.num_subcores
    nw = nc * ns
    b_per_w = B // nw
    mesh = plsc.VectorSubcoreMesh(core_axis_name="c", subcore_axis_name="s")

    @functools.partial(
        pl.kernel,
        mesh=mesh,
        out_type=jax.ShapeDtypeStruct((B, ROWP), jnp.float32),
        scratch_types=[
            pltpu.VMEM((b_per_w,), jnp.int32),
            pltpu.VMEM((b_per_w, ROWP), jnp.float32),
            pltpu.SemaphoreType.DMA,
        ],
        compiler_params=pltpu.CompilerParams(use_tc_tiling_on_sc=False),
    )
    def gather(table_hbm, idx_hbm, out_hbm, idx_v, rows_v, sem):
        wid = lax.axis_index("s") * nc + lax.axis_index("c")
        base = wid * b_per_w
        pltpu.sync_copy(idx_hbm.at[pl.ds(base, b_per_w)], idx_v)
        pltpu.async_copy(table_hbm.at[idx_v], rows_v, sem).wait()
        pltpu.sync_copy(rows_v, out_hbm.at[pl.ds(base, b_per_w)])

    return gather


def _mlp_body(cat_ref, u_ref, gt_ref, w1_ref, b1_ref, w2_ref, b2_ref,
              w3_ref, b3_ref, out_ref):
    u = u_ref[:, :EMB]
    cat = cat_ref[0, 0, :]
    col = lax.broadcasted_iota(jnp.int32, (BLK, GPAD), 1)
    onehot = (col == cat[:, None]).astype(jnp.float32)
    g = jnp.dot(onehot, gt_ref[...], preferred_element_type=jnp.float32)
    x = jnp.concatenate([u, g], axis=1)
    h = jnp.maximum(
        jnp.dot(x, w1_ref[...], preferred_element_type=jnp.float32)
        + b1_ref[...], 0.0)
    h = jnp.maximum(
        jnp.dot(h, w2_ref[...], preferred_element_type=jnp.float32)
        + b2_ref[...], 0.0)
    out = (jnp.dot(h, w3_ref[...], preferred_element_type=jnp.float32)
           + b3_ref[...])
    out_ref[...] = out.T


@functools.cache
def _build_mlp(interpret=False):
    full = lambda *shape: pl.BlockSpec(shape, lambda i: (0,) * len(shape))
    return pl.pallas_call(
        _mlp_body,
        grid=(GRID,),
        in_specs=[
            pl.BlockSpec((1, 1, BLK), lambda i: (i, 0, 0)),
            pl.BlockSpec((BLK, ROWP), lambda i: (i, 0)),
            full(GPAD, EMB),
            full(2 * EMB, 128), full(1, 128),
            full(128, 64), full(1, 64),
            full(64, EMB), full(1, EMB),
        ],
        out_specs=pl.BlockSpec((EMB, BLK), lambda i: (0, i)),
        out_shape=jax.ShapeDtypeStruct((EMB, B), jnp.float32),
        interpret=interpret,
    )


def kernel(customer_id, category_by_Gender, user_table, gender_table,
           W1, b1, W2, b2, W3, b3):
    cid = customer_id.astype(jnp.int32)
    cat = category_by_Gender.astype(jnp.int32).reshape(GRID, 1, BLK)
    table_p = jnp.pad(user_table, ((0, 0), (0, ROWP - EMB)))
    user_emb = _build_gather()(table_p, cid)
    gt_pad = jnp.pad(gender_table, ((0, GPAD - gender_table.shape[0]), (0, 0)))
    out_t = _build_mlp()(
        cat, user_emb, gt_pad,
        W1, b1.reshape(1, -1), W2, b2.reshape(1, -1), W3, b3.reshape(1, -1))
    return out_t.T
